# baseline (speedup 1.0000x reference)
import functools
import math

import jax
import jax.numpy as jnp
from jax import lax
from jax.experimental import pallas as pl
from jax.experimental.pallas import tpu as pltpu
from jax.experimental.pallas import tpu_sc as plsc

D_MODEL = 64
SCALE = math.sqrt(D_MODEL)

NUM_CORES = 2
NUM_SUBCORES = 16
NW = NUM_CORES * NUM_SUBCORES
LANES = 16


@functools.lru_cache(maxsize=None)
def _make_lookup(B, CH):
    n_per = B // NW
    n_chunks = n_per // CH

    mesh = plsc.VectorSubcoreMesh(core_axis_name="c", subcore_axis_name="s")

    @functools.partial(
        pl.kernel,
        out_type=jax.ShapeDtypeStruct((B, D_MODEL), jnp.float32),
        mesh=mesh,
        scratch_types=[
            pltpu.VMEM((CH,), jnp.int32),
            pltpu.VMEM((CH, D_MODEL), jnp.float32),
            pltpu.SemaphoreType.DMA,
        ],
        compiler_params=pltpu.CompilerParams(
            needs_layout_passes=False, has_side_effects=False
        ),
    )
    def lookup(x_hbm, emb_hbm, out_hbm, idx_s, rows_v, sem):
        wid = lax.axis_index("s") * NUM_CORES + lax.axis_index("c")
        base = wid * n_per

        def chunk_body(g, carry):
            cbase = base + g * CH
            pltpu.sync_copy(x_hbm.at[pl.ds(cbase, CH)], idx_s)

            def row_fire(gi, c):
                r0 = gi * LANES
                vec = idx_s[pl.ds(r0, LANES)]
                for l in range(LANES):
                    pltpu.make_async_copy(
                        emb_hbm.at[vec[l]], rows_v.at[r0 + l], sem
                    ).start()
                return c

            lax.fori_loop(0, CH // LANES, row_fire, 0)
            pltpu.make_async_copy(
                emb_hbm.at[pl.ds(0, CH)], rows_v, sem
            ).wait()

            def scale_body(r, c):
                for j in range(D_MODEL // LANES):
                    sl = pl.ds(j * LANES, LANES)
                    rows_v[r, sl] = rows_v[r, sl] * SCALE
                return c

            lax.fori_loop(0, CH, scale_body, 0, unroll=4)
            pltpu.sync_copy(rows_v, out_hbm.at[pl.ds(cbase, CH)])
            return carry

        lax.fori_loop(0, n_chunks, chunk_body, 0)

    return lookup


def kernel(x, emb):
    bsz, hist = x.shape
    B = bsz * hist
    xf = x.reshape(B).astype(jnp.int32)
    out = _make_lookup(B, 400)(xf, emb)
    return out.reshape(1, bsz, hist, D_MODEL)


# speedup vs baseline: 1.3284x; 1.3284x over previous
import functools
import math

import jax
import jax.experimental.layout
import jax.numpy as jnp
from jax import lax
from jax.experimental import pallas as pl
from jax.experimental.pallas import tpu as pltpu
from jax.experimental.pallas import tpu_sc as plsc

D_MODEL = 64
SCALE = math.sqrt(D_MODEL)

NUM_CORES = 2
NUM_SUBCORES = 16
NW = NUM_CORES * NUM_SUBCORES
LANES = 16


@functools.lru_cache(maxsize=None)
def _make_lookup(B, CH):
    n_per = B // NW
    n_chunks = n_per // CH

    mesh = plsc.VectorSubcoreMesh(core_axis_name="c", subcore_axis_name="s")

    @functools.partial(
        pl.kernel,
        out_type=jax.ShapeDtypeStruct((B, D_MODEL), jnp.float32),
        mesh=mesh,
        scratch_types=[
            pltpu.VMEM((CH,), jnp.int32),
            pltpu.VMEM((CH, D_MODEL), jnp.float32),
            pltpu.SemaphoreType.DMA,
        ],
        compiler_params=pltpu.CompilerParams(needs_layout_passes=False),
    )
    def lookup(x_hbm, emb_hbm, out_hbm, idx_s, rows_v, sem):
        wid = lax.axis_index("s") * NUM_CORES + lax.axis_index("c")
        base = wid * n_per

        def chunk_body(g, carry):
            cbase = base + g * CH
            pltpu.sync_copy(x_hbm.at[pl.ds(cbase, CH)], idx_s)

            def row_fire(gi, c):
                r0 = gi * LANES
                vec = idx_s[pl.ds(r0, LANES)]
                for l in range(LANES):
                    i = vec[l]
                    pltpu.make_async_copy(
                        emb_hbm.at[i >> 3, i & 7], rows_v.at[r0 + l], sem
                    ).start()
                return c

            lax.fori_loop(0, CH // LANES, row_fire, 0)
            pltpu.make_async_copy(
                out_hbm.at[pl.ds(0, CH)], rows_v, sem
            ).wait()

            def scale_body(r, c):
                for j in range(D_MODEL // LANES):
                    sl = pl.ds(j * LANES, LANES)
                    rows_v[r, sl] = rows_v[r, sl] * SCALE
                return c

            lax.fori_loop(0, CH, scale_body, 0, unroll=4)
            pltpu.sync_copy(rows_v, out_hbm.at[pl.ds(cbase, CH)])
            return carry

        lax.fori_loop(0, n_chunks, chunk_body, 0)

    return lookup


def kernel(x, emb):
    bsz, hist = x.shape
    B = bsz * hist
    xf = x.reshape(B).astype(jnp.int32)
    emb_rm = jax.experimental.layout.with_layout_constraint(
        emb,
        jax.experimental.layout.Layout((1, 0)),
    )
    emb3 = emb_rm.reshape(emb.shape[0] // 8, 8, D_MODEL)
    out = _make_lookup(B, 400)(xf, emb3)
    return out.reshape(1, bsz, hist, D_MODEL)


# speedup vs baseline: 1.3924x; 1.0482x over previous
import functools
import math

import jax
import jax.experimental.layout
import jax.numpy as jnp
from jax import lax
from jax.experimental import pallas as pl
from jax.experimental.pallas import tpu as pltpu
from jax.experimental.pallas import tpu_sc as plsc

D_MODEL = 64
SCALE = math.sqrt(D_MODEL)

NUM_CORES = 2
NUM_SUBCORES = 16
NW = NUM_CORES * NUM_SUBCORES
LANES = 16


@functools.lru_cache(maxsize=None)
def _make_lookup(B, CH):
    n_per = B // NW
    n_chunks = n_per // CH

    mesh = plsc.VectorSubcoreMesh(core_axis_name="c", subcore_axis_name="s")

    @functools.partial(
        pl.kernel,
        out_type=jax.ShapeDtypeStruct((B, D_MODEL), jnp.float32),
        mesh=mesh,
        scratch_types=[
            pltpu.VMEM((CH,), jnp.int32),
            pltpu.VMEM((CH,), jnp.int32),
            pltpu.VMEM((CH, D_MODEL), jnp.float32),
            pltpu.VMEM((CH, D_MODEL), jnp.float32),
            pltpu.SemaphoreType.DMA,
            pltpu.SemaphoreType.DMA,
        ],
        compiler_params=pltpu.CompilerParams(needs_layout_passes=False),
    )
    def lookup(x_hbm, emb_hbm, out_hbm, idx0, idx1, rows0, rows1, sem0, sem1):
        wid = lax.axis_index("s") * NUM_CORES + lax.axis_index("c")
        base = wid * n_per
        idx_b = (idx0, idx1)
        rows_b = (rows0, rows1)
        sem_b = (sem0, sem1)

        def fire(g, b):
            idx_s, rows_v, sem = idx_b[b], rows_b[b], sem_b[b]
            pltpu.sync_copy(x_hbm.at[pl.ds(base + g * CH, CH)], idx_s)

            def row_fire(gi, c):
                r0 = gi * LANES
                vec = idx_s[pl.ds(r0, LANES)]
                for l in range(LANES):
                    i = vec[l]
                    pltpu.make_async_copy(
                        emb_hbm.at[i >> 3, i & 7], rows_v.at[r0 + l], sem
                    ).start()
                return c

            lax.fori_loop(0, CH // LANES, row_fire, 0)

        def finish(g, b):
            rows_v, sem = rows_b[b], sem_b[b]
            pltpu.make_async_copy(out_hbm.at[pl.ds(0, CH)], rows_v, sem).wait()

            def scale_body(r, c):
                for j in range(D_MODEL // LANES):
                    sl = pl.ds(j * LANES, LANES)
                    rows_v[r, sl] = rows_v[r, sl] * SCALE
                return c

            lax.fori_loop(0, CH, scale_body, 0, unroll=4)
            pltpu.sync_copy(rows_v, out_hbm.at[pl.ds(base + g * CH, CH)])

        fire(0, 0)

        def pair_body(p, carry):
            g0 = 2 * p
            fire(g0 + 1, 1)
            finish(g0, 0)

            @pl.when(p + 1 < n_chunks // 2)
            def _():
                fire(g0 + 2, 0)

            finish(g0 + 1, 1)
            return carry

        lax.fori_loop(0, n_chunks // 2, pair_body, 0)

    return lookup


def kernel(x, emb):
    bsz, hist = x.shape
    B = bsz * hist
    xf = x.reshape(B).astype(jnp.int32)
    emb_rm = jax.experimental.layout.with_layout_constraint(
        emb,
        jax.experimental.layout.Layout((1, 0)),
    )
    emb3 = emb_rm.reshape(emb.shape[0] // 8, 8, D_MODEL)
    out = _make_lookup(B, 400)(xf, emb3)
    return out.reshape(1, bsz, hist, D_MODEL)
